# baseline (speedup 1.0000x reference)
import functools

import jax
import jax.numpy as jnp
from jax import lax
from jax.experimental import pallas as pl
from jax.experimental.pallas import tpu as pltpu
from jax.experimental.pallas import tpu_sc as plsc

_NC = 2
_NS = 16
_NW = _NC * _NS


def _sc_gather(ids, table, n_ch, ch, nbuf):
    V, D = table.shape
    bpw = n_ch * ch
    B = _NW * bpw
    wpr = ids.shape[1] // bpw
    mesh = plsc.VectorSubcoreMesh(core_axis_name="c", subcore_axis_name="s")
    n_groups = -(-n_ch // nbuf)
    exact = n_ch % nbuf == 0

    @functools.partial(
        pl.kernel,
        mesh=mesh,
        out_type=jax.ShapeDtypeStruct((B, D), jnp.float32),
        scratch_types=[
            pltpu.VMEM((bpw,), jnp.int32),
        ]
        + [pltpu.VMEM((ch, D), jnp.float32)] * nbuf
        + [pltpu.SemaphoreType.DMA] * (2 * nbuf),
    )
    def k(idx_hbm, table_hbm, out_hbm, idx_v, *rest):
        bufs = rest[:nbuf]
        gsem = rest[nbuf : 2 * nbuf]
        ssem = rest[2 * nbuf :]
        wid = lax.axis_index("s") * _NC + lax.axis_index("c")
        base = wid * bpw

        pltpu.sync_copy(idx_hbm.at[wid // wpr, pl.ds((wid % wpr) * bpw, bpw)], idx_v)

        def g_start(i, b):
            pltpu.async_copy(table_hbm.at[idx_v.at[pl.ds(i * ch, ch)]], bufs[b], gsem[b])

        def g_wait(b):
            pltpu.make_async_copy(
                table_hbm.at[idx_v.at[pl.ds(0, ch)]], bufs[b], gsem[b]
            ).wait()

        def s_start(i, b):
            pltpu.async_copy(bufs[b], out_hbm.at[pl.ds(base + i * ch, ch)], ssem[b])

        def s_wait(b):
            pltpu.make_async_copy(bufs[b], out_hbm.at[pl.ds(base, ch)], ssem[b]).wait()

        L = nbuf - 2
        for j in range(L):
            g_start(j, j)

        def loop_body(it, carry):
            c = it * nbuf
            for b in range(nbuf):
                i = c + b
                bj = (b + L) % nbuf

                def step(i=i, b=b, bj=bj):
                    g_wait(b)
                    s_start(i, b)

                    @pl.when(jnp.logical_and(i + L < n_ch, i >= nbuf - L))
                    def _():
                        s_wait(bj)
                        g_start(i + L, bj)

                    @pl.when(i + L < min(nbuf, n_ch))
                    def _():
                        g_start(i + L, bj)

                if exact:
                    step()
                else:
                    pl.when(i < n_ch)(step)

            return carry

        lax.fori_loop(0, n_groups, loop_body, 0)
        for b in range(nbuf):
            s_wait(b)

    return k(ids, table)


def _rope_tc(inv2, seq, hd):
    def body(if_ref, cos_ref, sin_ref, pos_ref):
        pos = lax.broadcasted_iota(jnp.int32, (seq, hd), 0).astype(jnp.float32)
        freqs = pos * if_ref[...]
        cos_ref[...] = jnp.cos(freqs)
        sin_ref[...] = jnp.sin(freqs)
        pos_ref[...] = lax.broadcasted_iota(jnp.int32, (1, seq), 1)

    return pl.pallas_call(
        body,
        out_shape=(
            jax.ShapeDtypeStruct((seq, hd), jnp.float32),
            jax.ShapeDtypeStruct((seq, hd), jnp.float32),
            jax.ShapeDtypeStruct((1, seq), jnp.int32),
        ),
    )(inv2)


def kernel(input_ids, table, inv_freq):
    batch, seq = input_ids.shape
    V, D = table.shape
    hd = 2 * inv_freq.shape[0]

    B = batch * seq
    ch = 8
    nbuf = 4
    n_ch = B // (_NW * ch)

    ids = input_ids.astype(jnp.int32)
    hidden = _sc_gather(ids, table, n_ch, ch, nbuf).reshape(batch, seq, D)

    inv2 = jnp.concatenate([inv_freq, inv_freq]).reshape(1, hd)
    cos, sin, position_ids = _rope_tc(inv2, seq, hd)
    return (hidden, position_ids, (cos[None], sin[None]))


# speedup vs baseline: 1.0051x; 1.0051x over previous
import functools

import jax
import jax.numpy as jnp
from jax import lax
from jax.experimental import pallas as pl
from jax.experimental.pallas import tpu as pltpu
from jax.experimental.pallas import tpu_sc as plsc

_NC = 2
_NS = 16
_NW = _NC * _NS


def _sc_gather(ids, table, n_ch, ch, nbuf):
    V, D = table.shape
    bpw = n_ch * ch
    B = _NW * bpw
    wpr = ids.shape[1] // bpw
    mesh = plsc.VectorSubcoreMesh(core_axis_name="c", subcore_axis_name="s")
    L = nbuf - 2

    @functools.partial(
        pl.kernel,
        mesh=mesh,
        out_type=jax.ShapeDtypeStruct((B, D), jnp.float32),
        scratch_types=[
            pltpu.VMEM((bpw,), jnp.int32),
            pltpu.VMEM((nbuf, ch, D), jnp.float32),
            pltpu.SemaphoreType.DMA((nbuf,)),
            pltpu.SemaphoreType.DMA((nbuf,)),
        ],
    )
    def k(idx_hbm, table_hbm, out_hbm, idx_v, bufs, gsem, ssem):
        wid = lax.axis_index("s") * _NC + lax.axis_index("c")
        base = wid * bpw

        pltpu.sync_copy(idx_hbm.at[wid // wpr, pl.ds((wid % wpr) * bpw, bpw)], idx_v)

        def g_start(i, b):
            pltpu.async_copy(
                table_hbm.at[idx_v.at[pl.ds(i * ch, ch)]], bufs.at[b], gsem.at[b]
            )

        def g_wait(b):
            pltpu.make_async_copy(
                table_hbm.at[idx_v.at[pl.ds(0, ch)]], bufs.at[b], gsem.at[b]
            ).wait()

        def s_start(i, b):
            pltpu.async_copy(
                bufs.at[b], out_hbm.at[pl.ds(base + i * ch, ch)], ssem.at[b]
            )

        def s_wait(b):
            pltpu.make_async_copy(
                bufs.at[b], out_hbm.at[pl.ds(base, ch)], ssem.at[b]
            ).wait()

        for j in range(L):
            g_start(j, j)

        def head(i, carry):
            b = lax.rem(i, nbuf)
            g_wait(b)
            s_start(i, b)
            g_start(i + L, lax.rem(i + L, nbuf))
            return carry

        def steady(i, carry):
            b = lax.rem(i, nbuf)
            bj = lax.rem(i + L, nbuf)
            g_wait(b)
            s_start(i, b)
            s_wait(bj)
            g_start(i + L, bj)
            return carry

        def tail(i, carry):
            b = lax.rem(i, nbuf)
            g_wait(b)
            s_start(i, b)
            return carry

        lax.fori_loop(0, nbuf - L, head, 0)
        lax.fori_loop(nbuf - L, n_ch - L, steady, 0)
        lax.fori_loop(n_ch - L, n_ch, tail, 0)
        for b in range(nbuf):
            s_wait(b)

    return k(ids, table)


def _rope_tc(inv2, seq, hd):
    def body(if_ref, cos_ref, sin_ref, pos_ref):
        pos = lax.broadcasted_iota(jnp.int32, (seq, hd), 0).astype(jnp.float32)
        freqs = pos * if_ref[...]
        cos_ref[...] = jnp.cos(freqs)
        sin_ref[...] = jnp.sin(freqs)
        pos_ref[...] = lax.broadcasted_iota(jnp.int32, (1, seq), 1)

    return pl.pallas_call(
        body,
        out_shape=(
            jax.ShapeDtypeStruct((seq, hd), jnp.float32),
            jax.ShapeDtypeStruct((seq, hd), jnp.float32),
            jax.ShapeDtypeStruct((1, seq), jnp.int32),
        ),
    )(inv2)


def kernel(input_ids, table, inv_freq):
    batch, seq = input_ids.shape
    V, D = table.shape
    hd = 2 * inv_freq.shape[0]

    B = batch * seq
    ch = 8
    nbuf = 4
    n_ch = B // (_NW * ch)

    ids = input_ids.astype(jnp.int32)
    hidden = _sc_gather(ids, table, n_ch, ch, nbuf).reshape(batch, seq, D)

    inv2 = jnp.concatenate([inv_freq, inv_freq]).reshape(1, hd)
    cos, sin, position_ids = _rope_tc(inv2, seq, hd)
    return (hidden, position_ids, (cos[None], sin[None]))


# speedup vs baseline: 1.0077x; 1.0025x over previous
import functools

import jax
import jax.numpy as jnp
from jax import lax
from jax.experimental import pallas as pl
from jax.experimental.pallas import tpu as pltpu
from jax.experimental.pallas import tpu_sc as plsc

_NC = 2
_NS = 16
_NW = _NC * _NS


def _sc_gather(ids, table, n_ch, ch, nbuf):
    V, D = table.shape
    bpw = n_ch * ch
    B = _NW * bpw
    wpr = ids.shape[1] // bpw
    mesh = plsc.VectorSubcoreMesh(core_axis_name="c", subcore_axis_name="s")
    L = nbuf - 2

    @functools.partial(
        pl.kernel,
        mesh=mesh,
        out_type=jax.ShapeDtypeStruct((B, D), jnp.float32),
        scratch_types=[
            pltpu.VMEM((bpw,), jnp.int32),
            pltpu.VMEM((nbuf, ch, D), jnp.float32),
            pltpu.SemaphoreType.DMA((nbuf,)),
            pltpu.SemaphoreType.DMA((nbuf,)),
        ],
    )
    def k(idx_hbm, table_hbm, out_hbm, idx_v, bufs, gsem, ssem):
        wid = lax.axis_index("s") * _NC + lax.axis_index("c")
        base = wid * bpw

        pltpu.sync_copy(idx_hbm.at[wid // wpr, pl.ds((wid % wpr) * bpw, bpw)], idx_v)

        def g_start(i, b):
            pltpu.async_copy(
                table_hbm.at[idx_v.at[pl.ds(i * ch, ch)]], bufs.at[b], gsem.at[b]
            )

        def g_wait(b):
            pltpu.make_async_copy(
                table_hbm.at[idx_v.at[pl.ds(0, ch)]], bufs.at[b], gsem.at[b]
            ).wait()

        def s_start(i, b):
            pltpu.async_copy(
                bufs.at[b], out_hbm.at[pl.ds(base + i * ch, ch)], ssem.at[b]
            )

        def s_wait(b):
            pltpu.make_async_copy(
                bufs.at[b], out_hbm.at[pl.ds(base, ch)], ssem.at[b]
            ).wait()

        for j in range(L):
            g_start(j, j)

        def head(i, carry):
            b = lax.rem(i, nbuf)
            g_wait(b)
            s_start(i, b)
            g_start(i + L, lax.rem(i + L, nbuf))
            return carry

        def steady(i, carry):
            b = lax.rem(i, nbuf)
            bj = lax.rem(i + L, nbuf)
            g_wait(b)
            s_start(i, b)
            s_wait(bj)
            g_start(i + L, bj)
            return carry

        def tail(i, carry):
            b = lax.rem(i, nbuf)
            g_wait(b)
            s_start(i, b)
            return carry

        lax.fori_loop(0, nbuf - L, head, 0)
        lax.fori_loop(nbuf - L, n_ch - L, steady, 0)
        lax.fori_loop(n_ch - L, n_ch, tail, 0)
        for b in range(nbuf):
            s_wait(b)

    return k(ids, table)


def _rope_tc(inv2, seq, hd):
    def body(if_ref, cos_ref, sin_ref, pos_ref):
        pos = lax.broadcasted_iota(jnp.int32, (seq, hd), 0).astype(jnp.float32)
        freqs = pos * if_ref[...]
        cos_ref[...] = jnp.cos(freqs)
        sin_ref[...] = jnp.sin(freqs)
        pos_ref[...] = lax.broadcasted_iota(jnp.int32, (1, seq), 1)

    return pl.pallas_call(
        body,
        out_shape=(
            jax.ShapeDtypeStruct((seq, hd), jnp.float32),
            jax.ShapeDtypeStruct((seq, hd), jnp.float32),
            jax.ShapeDtypeStruct((1, seq), jnp.int32),
        ),
    )(inv2)


def kernel(input_ids, table, inv_freq):
    batch, seq = input_ids.shape
    V, D = table.shape
    hd = 2 * inv_freq.shape[0]

    B = batch * seq
    ch = 8
    nbuf = 6
    n_ch = B // (_NW * ch)

    ids = input_ids.astype(jnp.int32)
    hidden = _sc_gather(ids, table, n_ch, ch, nbuf).reshape(batch, seq, D)

    inv2 = jnp.concatenate([inv_freq, inv_freq]).reshape(1, hd)
    cos, sin, position_ids = _rope_tc(inv2, seq, hd)
    return (hidden, position_ids, (cos[None], sin[None]))
